# baseline (speedup 1.0000x reference)
import jax
import jax.numpy as jnp
from jax import lax
from jax.experimental import pallas as pl
from jax.experimental.pallas import tpu as pltpu
from jax.experimental.pallas import tpu_sc as plsc

R = 128
C = 32768
NC = 2
NS = 16
NW = NC * NS
RPW = R // NW
L = 16
NV = C // L


def _shuffle(v, idx):
    dnums = lax.GatherDimensionNumbers(
        offset_dims=(), collapsed_slice_dims=(0,), start_index_map=(0,))
    return lax.gather(v, idx[:, None], dnums, (1,),
                      mode=lax.GatherScatterMode.PROMISE_IN_BOUNDS)


def _scan_row(buf):
    lane = lax.iota(jnp.int32, L)

    def it(i, carry):
        bv, bi = carry
        v = buf[pl.ds(i * L, L)]
        gt = v > bv
        iv = lane + i * L
        return jnp.where(gt, v, bv), jnp.where(gt, iv, bi)

    init = (jnp.full((L,), -jnp.inf, jnp.float32), jnp.zeros((L,), jnp.int32))
    bv, bi = lax.fori_loop(0, NV, it, init, unroll=8)

    for s in (8, 4, 2, 1):
        perm = lane ^ s
        ov = _shuffle(bv, perm)
        oi = _shuffle(bi, perm)
        better = (ov > bv) | ((ov == bv) & (oi < bi))
        bv = jnp.where(better, ov, bv)
        bi = jnp.where(better, oi, bi)
    return bi


def _body(x_hbm, out_hbm, buf0, buf1, res, sem0, sem1):
    cid = lax.axis_index("c")
    sid = lax.axis_index("s")
    wid = sid * NC + cid
    r0 = wid * RPW

    bufs = (buf0, buf1)
    sems = (sem0, sem1)

    cps = [pltpu.async_copy(x_hbm.at[r0 + j], bufs[j], sems[j])
           for j in range(2)]
    for j in range(RPW):
        b = j % 2
        cps[b].wait()
        amin = _scan_row(bufs[b])
        if j + 2 < RPW:
            cps[b] = pltpu.async_copy(x_hbm.at[r0 + j + 2], bufs[b], sems[b])
        res[j, :] = amin

    pltpu.sync_copy(res, out_hbm.at[wid])


@jax.jit
def _argmax_sc(x):
    mesh = plsc.VectorSubcoreMesh(core_axis_name="c", subcore_axis_name="s")
    k = pl.kernel(
        _body,
        mesh=mesh,
        out_type=jax.ShapeDtypeStruct((NW, RPW, L), jnp.int32),
        scratch_types=[
            pltpu.VMEM((C,), jnp.float32),
            pltpu.VMEM((C,), jnp.float32),
            pltpu.VMEM((RPW, L), jnp.int32),
            pltpu.SemaphoreType.DMA,
            pltpu.SemaphoreType.DMA,
        ],
    )
    out = k(x)
    return out.reshape(R, L)[:, 0]


def kernel(x):
    return _argmax_sc(x)
